# Initial kernel scaffold; baseline (speedup 1.0000x reference)
import jax
import jax.numpy as jnp
from jax.experimental import pallas as pl


def kernel(sem_ids, token_type_ids, seq_mask, sem_ids_fut, token_type_ids_fut, table):
    raise NotImplementedError("write your pallas kernel here")


# speedup vs baseline: 1.8739x; 1.8739x over previous
import functools

import jax
import jax.numpy as jnp
from jax import lax
from jax.experimental import pallas as pl
from jax.experimental.pallas import tpu as pltpu
from jax.experimental.pallas import tpu_sc as plsc

NUM_EMB = 100000
SEM_DIM = 4
EMB_DIM = 64
PAD = NUM_EMB * SEM_DIM

B, L, LF = 4096, 200, 4
NSEQ = B * L
NFUT = B * LF

NC, NS, LANES = 2, 16, 16
NW = NC * NS

SEQ_PER_W = NSEQ // NW
FUT_PER_W = NFUT // NW
CHUNK = 128
SEQ_CHUNKS = SEQ_PER_W // CHUNK
FUT_CHUNKS = FUT_PER_W // CHUNK
VEC_PER_CHUNK = CHUNK // LANES


def _compute_idx_chunk(sem_v, tt_v, msk_v, idx_v, j, use_mask):
  for u in range(VEC_PER_CHUNK):
    off = j * CHUNK + u * LANES
    s = sem_v[pl.ds(off, LANES)]
    t = tt_v[pl.ds(off, LANES)]
    tc = jnp.clip(t, 0, SEM_DIM - 1)
    idx = tc * NUM_EMB + s
    valid = (s >= 0) & (s < NUM_EMB)
    idx = jnp.where(valid, idx, PAD)
    if use_mask:
      m = msk_v[pl.ds(off, LANES)]
      idx = jnp.where(m != 0, idx, PAD)
    idx_v[j, pl.ds(u * LANES, LANES)] = idx


def _sc_body(sem_h, tt_h, msk_h, semf_h, ttf_h, table_h,
             out_seq_h, out_fut_h,
             sem_v, tt_v, msk_v, idx_v, rows_v, dsem):
  wid = lax.axis_index("s") * NC + lax.axis_index("c")

  base = wid * SEQ_PER_W
  pltpu.sync_copy(sem_h.at[pl.ds(base, SEQ_PER_W)], sem_v)
  pltpu.sync_copy(tt_h.at[pl.ds(base, SEQ_PER_W)], tt_v)
  pltpu.sync_copy(msk_h.at[pl.ds(base, SEQ_PER_W)], msk_v)

  def idx_loop(j, _):
    _compute_idx_chunk(sem_v, tt_v, msk_v, idx_v, j, use_mask=True)
    return _
  lax.fori_loop(0, SEQ_CHUNKS, idx_loop, 0)

  def gather_loop(j, _):
    pltpu.async_copy(table_h.at[idx_v.at[j]], rows_v, dsem).wait()
    pltpu.sync_copy(rows_v, out_seq_h.at[pl.ds(base + j * CHUNK, CHUNK)])
    return _
  lax.fori_loop(0, SEQ_CHUNKS, gather_loop, 0)

  basef = wid * FUT_PER_W
  pltpu.sync_copy(semf_h.at[pl.ds(basef, FUT_PER_W)],
                  sem_v.at[pl.ds(0, FUT_PER_W)])
  pltpu.sync_copy(ttf_h.at[pl.ds(basef, FUT_PER_W)],
                  tt_v.at[pl.ds(0, FUT_PER_W)])

  def idxf_loop(j, _):
    _compute_idx_chunk(sem_v, tt_v, msk_v, idx_v, j, use_mask=False)
    return _
  lax.fori_loop(0, FUT_CHUNKS, idxf_loop, 0)

  def gatherf_loop(j, _):
    pltpu.async_copy(table_h.at[idx_v.at[j]], rows_v, dsem).wait()
    pltpu.sync_copy(rows_v, out_fut_h.at[pl.ds(basef + j * CHUNK, CHUNK)])
    return _
  lax.fori_loop(0, FUT_CHUNKS, gatherf_loop, 0)


@functools.partial(jax.jit, static_argnums=())
def _run(sem_flat, tt_flat, msk_flat, semf_flat, ttf_flat, table):
  mesh = plsc.VectorSubcoreMesh(core_axis_name="c", subcore_axis_name="s",
                                num_cores=NC, num_subcores=NS)
  f = pl.kernel(
      _sc_body,
      out_type=[
          jax.ShapeDtypeStruct((NSEQ, EMB_DIM), jnp.float32),
          jax.ShapeDtypeStruct((NFUT, EMB_DIM), jnp.float32),
      ],
      mesh=mesh,
      scratch_types=[
          pltpu.VMEM((SEQ_PER_W,), jnp.int32),
          pltpu.VMEM((SEQ_PER_W,), jnp.int32),
          pltpu.VMEM((SEQ_PER_W,), jnp.int32),
          pltpu.VMEM((SEQ_CHUNKS, CHUNK), jnp.int32),
          pltpu.VMEM((CHUNK, EMB_DIM), jnp.float32),
          pltpu.SemaphoreType.DMA,
      ],
      compiler_params=pltpu.CompilerParams(use_tc_tiling_on_sc=False),
  )
  return f(sem_flat, tt_flat, msk_flat, semf_flat, ttf_flat, table)


def kernel(sem_ids, token_type_ids, seq_mask, sem_ids_fut, token_type_ids_fut,
           table):
  sem_flat = sem_ids.reshape(-1).astype(jnp.int32)
  tt_flat = token_type_ids.reshape(-1).astype(jnp.int32)
  msk_flat = seq_mask.reshape(-1).astype(jnp.int32)
  semf_flat = sem_ids_fut.reshape(-1).astype(jnp.int32)
  ttf_flat = token_type_ids_fut.reshape(-1).astype(jnp.int32)
  out_seq, out_fut = _run(sem_flat, tt_flat, msk_flat, semf_flat, ttf_flat,
                          table.astype(jnp.float32))
  return (out_seq.reshape(B, L, EMB_DIM), out_fut.reshape(B, LF, EMB_DIM))
